# baseline (speedup 1.0000x reference)
import functools

import jax
import jax.numpy as jnp
from jax import lax
from jax.experimental import pallas as pl
from jax.experimental.pallas import tpu as pltpu
from jax.experimental.pallas import tpu_sc as plsc

NUM_POS = 4096
NUM_DIM = 1024
BATCH = 4

_NC = 2
_NS = 16
_NW = _NC * _NS
_ROWS_PER_W = NUM_POS // _NW
_CHUNK = 64

_mesh = plsc.VectorSubcoreMesh(core_axis_name="c", subcore_axis_name="s")


@functools.partial(
    pl.kernel,
    mesh=_mesh,
    out_type=jax.ShapeDtypeStruct((BATCH, NUM_POS, NUM_DIM), jnp.float32),
    scratch_types=[pltpu.VMEM((_CHUNK, NUM_DIM), jnp.float32)],
)
def _broadcast_table(w_hbm, out_hbm, buf):
    wid = lax.axis_index("s") * _NC + lax.axis_index("c")
    base = wid * _ROWS_PER_W
    for chunk in range(_ROWS_PER_W // _CHUNK):
        start = base + chunk * _CHUNK
        pltpu.sync_copy(w_hbm.at[pl.ds(start, _CHUNK)], buf)
        for b in range(BATCH):
            pltpu.sync_copy(buf, out_hbm.at[b, pl.ds(start, _CHUNK)])


def kernel(x, w):
    del x
    return _broadcast_table(w)
